# baseline (speedup 1.0000x reference)
import functools

import jax
import jax.numpy as jnp
from jax.experimental import pallas as pl
from jax.experimental.pallas import tpu as pltpu

DM = 1024
NE = 16
ES = 128
TB = 512


def _moe_body(x_ref, wgt_ref, k_hbm, v_hbm, o_ref,
              k3_ref, v3_ref, km_ref, vm_ref, sem_k, sem_v):
    first = pl.program_id(0) == 0

    @pl.when(first)
    def _():
        pltpu.make_async_copy(k_hbm, k3_ref, sem_k).start()
        pltpu.make_async_copy(v_hbm, v3_ref, sem_v).start()

    xb = x_ref[...]
    logits = jax.lax.dot_general(
        xb, wgt_ref[...], (((1,), (1,)), ((), ())),
        preferred_element_type=jnp.float32)
    sel = jax.nn.sigmoid(logits)
    lane = jax.lax.broadcasted_iota(jnp.int32, (TB, NE), 1)
    m1 = jnp.max(logits, axis=1, keepdims=True)
    a1 = jnp.min(jnp.where(logits == m1, lane, NE), axis=1, keepdims=True)
    hot1 = lane == a1
    l2 = jnp.where(hot1, -jnp.inf, logits)
    m2 = jnp.max(l2, axis=1, keepdims=True)
    a2 = jnp.min(jnp.where(l2 == m2, lane, NE), axis=1, keepdims=True)
    gate = sel * (hot1 | (lane == a2)).astype(jnp.float32)

    @pl.when(first)
    def _():
        pltpu.make_async_copy(k_hbm, k3_ref, sem_k).wait()
        for e in range(NE):
            km_ref[:, e * ES:(e + 1) * ES] = k3_ref[e].astype(jnp.bfloat16)

    scores = jnp.dot(xb.astype(jnp.bfloat16), km_ref[...],
                     preferred_element_type=jnp.float32)

    @pl.when(first)
    def _():
        pltpu.make_async_copy(v_hbm, v3_ref, sem_v).wait()
        for e in range(NE):
            vm_ref[e * ES:(e + 1) * ES, :] = v3_ref[e].astype(jnp.bfloat16)
    h = jnp.concatenate(
        [jnp.maximum(scores[:, e * ES:(e + 1) * ES], 0.0) * gate[:, e:e + 1]
         for e in range(NE)], axis=1)
    o_ref[...] = jnp.dot(h.astype(jnp.bfloat16), vm_ref[...],
                         preferred_element_type=jnp.float32)


@jax.jit
def kernel(x, w_gate, keys, values):
    B, S, D = x.shape
    xf = x.reshape(-1, D)
    n = xf.shape[0]
    grid = (n // TB,)
    out = pl.pallas_call(
        _moe_body,
        grid=grid,
        in_specs=[
            pl.BlockSpec((TB, D), lambda i: (i, 0)),
            pl.BlockSpec((NE, D), lambda i: (0, 0)),
            pl.BlockSpec(memory_space=pltpu.MemorySpace.HBM),
            pl.BlockSpec(memory_space=pltpu.MemorySpace.HBM),
        ],
        out_specs=pl.BlockSpec((TB, D), lambda i: (i, 0)),
        out_shape=jax.ShapeDtypeStruct((n, D), jnp.float32),
        scratch_shapes=[
            pltpu.VMEM((NE, D, ES), jnp.float32),
            pltpu.VMEM((NE, ES, D), jnp.float32),
            pltpu.VMEM((D, NE * ES), jnp.bfloat16),
            pltpu.VMEM((NE * ES, D), jnp.bfloat16),
            pltpu.SemaphoreType.DMA,
            pltpu.SemaphoreType.DMA,
        ],
        compiler_params=pltpu.CompilerParams(
            dimension_semantics=("arbitrary",),
        ),
    )(xf, w_gate, keys, values)
    return out.reshape(B, S, D)


# speedup vs baseline: 1.1086x; 1.1086x over previous
import functools

import jax
import jax.numpy as jnp
from jax.experimental import pallas as pl
from jax.experimental.pallas import tpu as pltpu

DM = 1024
NE = 16
ES = 128
TB = 512


def _moe_body(x_ref, wgt_ref, k_ref, v_ref, o_ref, km_ref, vm_ref):
    @pl.when(pl.program_id(0) == 0)
    def _():
        for e in range(NE):
            km_ref[:, e * ES:(e + 1) * ES] = k_ref[e].astype(jnp.bfloat16)
            vm_ref[e * ES:(e + 1) * ES, :] = v_ref[e].astype(jnp.bfloat16)

    xb = x_ref[...]
    logits = jax.lax.dot_general(
        xb, wgt_ref[...], (((1,), (1,)), ((), ())),
        preferred_element_type=jnp.float32)
    sel = jax.nn.sigmoid(logits)
    lane = jax.lax.broadcasted_iota(jnp.int32, (TB, NE), 1)
    m1 = jnp.max(logits, axis=1, keepdims=True)
    a1 = jnp.min(jnp.where(logits == m1, lane, NE), axis=1, keepdims=True)
    hot1 = lane == a1
    l2 = jnp.where(hot1, -jnp.inf, logits)
    m2 = jnp.max(l2, axis=1, keepdims=True)
    a2 = jnp.min(jnp.where(l2 == m2, lane, NE), axis=1, keepdims=True)
    gate = sel * (hot1 | (lane == a2)).astype(jnp.float32)
    scores = jnp.dot(xb.astype(jnp.bfloat16), km_ref[...],
                     preferred_element_type=jnp.float32)
    h = jnp.concatenate(
        [jnp.maximum(scores[:, e * ES:(e + 1) * ES], 0.0) * gate[:, e:e + 1]
         for e in range(NE)], axis=1)
    o_ref[...] = jnp.dot(h.astype(jnp.bfloat16), vm_ref[...],
                         preferred_element_type=jnp.float32)


@jax.jit
def kernel(x, w_gate, keys, values):
    B, S, D = x.shape
    xf = x.reshape(-1, D)
    n = xf.shape[0]
    grid = (n // TB,)
    out = pl.pallas_call(
        _moe_body,
        grid=grid,
        in_specs=[
            pl.BlockSpec((TB, D), lambda i: (i, 0)),
            pl.BlockSpec((NE, D), lambda i: (0, 0)),
            pl.BlockSpec((NE, D, ES), lambda i: (0, 0, 0)),
            pl.BlockSpec((NE, ES, D), lambda i: (0, 0, 0)),
        ],
        out_specs=pl.BlockSpec((TB, D), lambda i: (i, 0)),
        out_shape=jax.ShapeDtypeStruct((n, D), jnp.float32),
        scratch_shapes=[
            pltpu.VMEM((D, NE * ES), jnp.bfloat16),
            pltpu.VMEM((NE * ES, D), jnp.bfloat16),
        ],
        compiler_params=pltpu.CompilerParams(
            dimension_semantics=("arbitrary",),
        ),
    )(xf, w_gate, keys, values)
    return out.reshape(B, S, D)
